# baseline (speedup 1.0000x reference)
import functools

import jax
import jax.numpy as jnp
from jax import lax
from jax.experimental import pallas as pl
from jax.experimental.pallas import tpu as pltpu
from jax.experimental.pallas import tpu_sc as plsc

NC = 2
NS = 16
CHUNK = 128


def _row_chunks(n_rows):
    offs = [t * CHUNK for t in range(n_rows // CHUNK)]
    if n_rows % CHUNK:
        offs.append(n_rows - CHUNK)
    return offs


def _sc_agg_body(n_chunks, n_rows_per_sub,
                 x2, src2, dst, z128,
                 agg_out,
                 gidx_v, dst_v, rows_v, acc):
    c = lax.axis_index("c")
    s = lax.axis_index("s")
    r0 = s * n_rows_per_sub
    offs = _row_chunks(n_rows_per_sub)

    pltpu.sync_copy(z128, rows_v)
    for off in offs:
        pltpu.sync_copy(rows_v, acc.at[pl.ds(r0 + off, CHUNK)])
    plsc.subcore_barrier()

    e0 = s * (n_chunks * CHUNK)

    def body(i, carry):
        base = e0 + i * CHUNK
        pltpu.sync_copy(src2.at[c, pl.ds(base, CHUNK)], gidx_v)
        pltpu.sync_copy(dst.at[pl.ds(base, CHUNK)], dst_v)
        pltpu.sync_copy(x2.at[gidx_v], rows_v)
        pltpu.sync_copy(rows_v, acc.at[dst_v], add=True)
        return carry

    lax.fori_loop(0, n_chunks, body, 0)
    plsc.subcore_barrier()

    for off in offs:
        pltpu.sync_copy(acc.at[pl.ds(r0 + off, CHUNK)], rows_v)
        pltpu.sync_copy(rows_v, agg_out.at[c, pl.ds(r0 + off, CHUNK)])


def _sc_deg_body(n_chunks_half, n_rows_per_sub,
                 dst32, z128, ones128,
                 deg_out,
                 dst_all, rows_v, acc):
    c = lax.axis_index("c")
    s = lax.axis_index("s")
    r0 = s * n_rows_per_sub
    offs = _row_chunks(n_rows_per_sub)

    pltpu.sync_copy(dst32.at[c, s], dst_all)
    pltpu.sync_copy(z128, rows_v)
    for off in offs:
        pltpu.sync_copy(rows_v, acc.at[pl.ds(r0 + off, CHUNK)])
    pltpu.sync_copy(ones128, rows_v)
    plsc.subcore_barrier()

    def body(i, carry):
        pltpu.sync_copy(rows_v, acc.at[dst_all.at[i]], add=True)
        return carry

    lax.fori_loop(0, n_chunks_half, body, 0)
    plsc.subcore_barrier()

    for off in offs:
        pltpu.sync_copy(acc.at[pl.ds(r0 + off, CHUNK)], rows_v)
        pltpu.sync_copy(rows_v, deg_out.at[c, pl.ds(r0 + off, CHUNK)])


def _tc_body(a0_ref, a1_ref, d0_ref, d1_ref, x_ref, w_ref, b_ref, g_ref,
             be_ref, o_ref):
    deg = jnp.maximum(d0_ref[0][:, 0:1] + d1_ref[0][:, 0:1], 1.0)
    a0 = a0_ref[0] / deg
    a1 = a1_ref[0] / deg
    h = lax.dot_general(a0, w_ref[0:128, :], (((1,), (0,)), ((), ())),
                        precision=lax.Precision.HIGHEST,
                        preferred_element_type=jnp.float32)
    h = h + lax.dot_general(a1, w_ref[128:256, :], (((1,), (0,)), ((), ())),
                            precision=lax.Precision.HIGHEST,
                            preferred_element_type=jnp.float32)
    y = x_ref[...] + h + b_ref[...]
    mu = jnp.mean(y, axis=1, keepdims=True)
    yc = y - mu
    var = jnp.mean(yc * yc, axis=1, keepdims=True)
    o_ref[...] = yc * lax.rsqrt(var + 1e-5) * g_ref[...] + be_ref[...]


@jax.jit
def kernel(x, edge_index, W, b, gamma, beta):
    N, D = x.shape
    E = edge_index.shape[1]
    assert D == 256
    HALF = D // 2

    src = edge_index[0]
    dst = edge_index[1]

    grp = 2 * NS * CHUNK * 2
    e_pad = -(-E // grp) * grp
    if e_pad != E:
        pad = e_pad - E
        src = jnp.concatenate([src, jnp.zeros((pad,), jnp.int32)])
        dst = jnp.concatenate([dst, jnp.full((pad,), N, jnp.int32)])
    n_chunks = e_pad // (NS * CHUNK)
    n_chunks_half = e_pad // (2 * NS * CHUNK)

    x2 = x.reshape(N * 2, HALF)
    src2 = jnp.stack([src * 2, src * 2 + 1])
    dst32 = dst.reshape(2, NS, n_chunks_half, CHUNK)

    n_rows_per_sub = -(-N // (NS * 8)) * 8
    n_pad = n_rows_per_sub * NS
    assert n_pad > N

    z128 = jnp.zeros((CHUNK, HALF), jnp.float32)
    ones128 = jnp.ones((CHUNK, HALF), jnp.float32)

    mesh = plsc.VectorSubcoreMesh(core_axis_name="c", subcore_axis_name="s")
    sc_agg = pl.kernel(
        functools.partial(_sc_agg_body, n_chunks, n_rows_per_sub),
        out_type=jax.ShapeDtypeStruct((2, n_pad, HALF), jnp.float32),
        mesh=mesh,
        scratch_types=[
            pltpu.VMEM((CHUNK,), jnp.int32),
            pltpu.VMEM((CHUNK,), jnp.int32),
            pltpu.VMEM((CHUNK, HALF), jnp.float32),
            pltpu.VMEM_SHARED((n_pad, HALF), jnp.float32),
        ],
    )
    agg2 = sc_agg(x2, src2, dst, z128)

    sc_deg = pl.kernel(
        functools.partial(_sc_deg_body, n_chunks_half, n_rows_per_sub),
        out_type=jax.ShapeDtypeStruct((2, n_pad, HALF), jnp.float32),
        mesh=mesh,
        scratch_types=[
            pltpu.VMEM((n_chunks_half, CHUNK), jnp.int32),
            pltpu.VMEM((CHUNK, HALF), jnp.float32),
            pltpu.VMEM_SHARED((n_pad, HALF), jnp.float32),
        ],
    )
    deg2 = sc_deg(dst32, z128, ones128)

    BN = 512
    grid = -(-N // BN)
    out = pl.pallas_call(
        _tc_body,
        grid=(grid,),
        in_specs=[
            pl.BlockSpec((1, BN, HALF), lambda i: (0, i, 0)),
            pl.BlockSpec((1, BN, HALF), lambda i: (1, i, 0)),
            pl.BlockSpec((1, BN, HALF), lambda i: (0, i, 0)),
            pl.BlockSpec((1, BN, HALF), lambda i: (1, i, 0)),
            pl.BlockSpec((BN, D), lambda i: (i, 0)),
            pl.BlockSpec((D, D), lambda i: (0, 0)),
            pl.BlockSpec((1, D), lambda i: (0, 0)),
            pl.BlockSpec((1, D), lambda i: (0, 0)),
            pl.BlockSpec((1, D), lambda i: (0, 0)),
        ],
        out_specs=pl.BlockSpec((BN, D), lambda i: (i, 0)),
        out_shape=jax.ShapeDtypeStruct((N, D), jnp.float32),
    )(agg2, agg2, deg2, deg2, x, W, b.reshape(1, D), gamma.reshape(1, D),
      beta.reshape(1, D))
    return out


# speedup vs baseline: 1.1033x; 1.1033x over previous
import functools

import jax
import jax.numpy as jnp
from jax import lax
from jax.experimental import pallas as pl
from jax.experimental.pallas import tpu as pltpu
from jax.experimental.pallas import tpu_sc as plsc

NC = 2
NS = 16
CHUNK = 128


def _row_chunks(n_rows):
    offs = [t * CHUNK for t in range(n_rows // CHUNK)]
    if n_rows % CHUNK:
        offs.append(n_rows - CHUNK)
    return offs


def _sc_agg_body(n_chunks, n_rows_per_sub,
                 x2, src2, dst, z128,
                 agg_out,
                 gidx_v, dst_v, rows_v, acc):
    c = lax.axis_index("c")
    s = lax.axis_index("s")
    r0 = s * n_rows_per_sub
    offs = _row_chunks(n_rows_per_sub)

    pltpu.sync_copy(z128, rows_v)
    for off in offs:
        pltpu.sync_copy(rows_v, acc.at[pl.ds(r0 + off, CHUNK)])
    plsc.subcore_barrier()

    e0 = s * (n_chunks * CHUNK)

    def body(i, carry):
        base = e0 + i * CHUNK
        pltpu.sync_copy(src2.at[c, pl.ds(base, CHUNK)], gidx_v)
        pltpu.sync_copy(dst.at[pl.ds(base, CHUNK)], dst_v)
        pltpu.sync_copy(x2.at[gidx_v], rows_v)
        pltpu.sync_copy(rows_v, acc.at[dst_v], add=True)
        return carry

    lax.fori_loop(0, n_chunks, body, 0)
    plsc.subcore_barrier()

    for off in offs:
        pltpu.sync_copy(acc.at[pl.ds(r0 + off, CHUNK)], rows_v)
        pltpu.sync_copy(rows_v, agg_out.at[c, pl.ds(r0 + off, CHUNK)])


def _sc_deg_body(n_chunks, n_rows_per_sub,
                 dst, z128, ones128,
                 deg_out,
                 dst_v, rows_v, ones_v, acc):
    s = lax.axis_index("s")
    r0 = s * n_rows_per_sub
    offs = _row_chunks(n_rows_per_sub)

    pltpu.sync_copy(z128, rows_v)
    pltpu.sync_copy(ones128, ones_v)
    for off in offs:
        pltpu.sync_copy(rows_v, acc.at[pl.ds(r0 + off, CHUNK)])
    plsc.subcore_barrier()

    e0 = s * (n_chunks * CHUNK)

    def body(i, carry):
        base = e0 + i * CHUNK
        pltpu.sync_copy(dst.at[pl.ds(base, CHUNK)], dst_v)
        pltpu.sync_copy(ones_v, acc.at[dst_v], add=True)
        return carry

    lax.fori_loop(0, n_chunks, body, 0)
    plsc.subcore_barrier()

    for off in offs:
        pltpu.sync_copy(acc.at[pl.ds(r0 + off, CHUNK)], rows_v)
        pltpu.sync_copy(rows_v, deg_out.at[pl.ds(r0 + off, CHUNK)])


def _tc_body(a0_ref, a1_ref, deg_ref, x_ref, w_ref, b_ref, g_ref, be_ref,
             o_ref):
    deg = jnp.maximum(deg_ref[:, 0:1], 1.0)
    a0 = a0_ref[0] / deg
    a1 = a1_ref[0] / deg
    h = lax.dot_general(a0, w_ref[0:128, :], (((1,), (0,)), ((), ())),
                        precision=lax.Precision.HIGHEST,
                        preferred_element_type=jnp.float32)
    h = h + lax.dot_general(a1, w_ref[128:256, :], (((1,), (0,)), ((), ())),
                            precision=lax.Precision.HIGHEST,
                            preferred_element_type=jnp.float32)
    y = x_ref[...] + h + b_ref[...]
    mu = jnp.mean(y, axis=1, keepdims=True)
    yc = y - mu
    var = jnp.mean(yc * yc, axis=1, keepdims=True)
    o_ref[...] = yc * lax.rsqrt(var + 1e-5) * g_ref[...] + be_ref[...]


@jax.jit
def kernel(x, edge_index, W, b, gamma, beta):
    N, D = x.shape
    E = edge_index.shape[1]
    assert D == 256
    HALF = D // 2

    src = edge_index[0]
    dst = edge_index[1]

    per_sub = -(-E // (NS * CHUNK)) * CHUNK
    n_chunks = per_sub // CHUNK
    e_pad = per_sub * NS
    if e_pad != E:
        pad = e_pad - E
        src = jnp.concatenate([src, jnp.zeros((pad,), jnp.int32)])
        dst = jnp.concatenate([dst, jnp.full((pad,), N, jnp.int32)])

    x2 = x.reshape(N * 2, HALF)
    src2 = jnp.stack([src * 2, src * 2 + 1])
    n_rows_per_sub = -(-N // (NS * 8)) * 8
    n_pad = n_rows_per_sub * NS
    assert n_pad > N

    z128 = jnp.zeros((CHUNK, HALF), jnp.float32)

    mesh = plsc.VectorSubcoreMesh(core_axis_name="c", subcore_axis_name="s")
    sc_agg = pl.kernel(
        functools.partial(_sc_agg_body, n_chunks, n_rows_per_sub),
        out_type=jax.ShapeDtypeStruct((2, n_pad, HALF), jnp.float32),
        mesh=mesh,
        scratch_types=[
            pltpu.VMEM((CHUNK,), jnp.int32),
            pltpu.VMEM((CHUNK,), jnp.int32),
            pltpu.VMEM((CHUNK, HALF), jnp.float32),
            pltpu.VMEM_SHARED((n_pad, HALF), jnp.float32),
        ],
    )
    agg2 = sc_agg(x2, src2, dst, z128)

    ones128 = jnp.ones((CHUNK, HALF), jnp.float32)
    sc_deg = pl.kernel(
        functools.partial(_sc_deg_body, n_chunks, n_rows_per_sub),
        out_type=jax.ShapeDtypeStruct((n_pad, HALF), jnp.float32),
        mesh=mesh,
        scratch_types=[
            pltpu.VMEM((CHUNK,), jnp.int32),
            pltpu.VMEM((CHUNK, HALF), jnp.float32),
            pltpu.VMEM((CHUNK, HALF), jnp.float32),
            pltpu.VMEM_SHARED((n_pad, HALF), jnp.float32),
        ],
    )
    deg2 = sc_deg(dst, z128, ones128)

    BN = 512
    grid = -(-N // BN)
    out = pl.pallas_call(
        _tc_body,
        grid=(grid,),
        in_specs=[
            pl.BlockSpec((1, BN, HALF), lambda i: (0, i, 0)),
            pl.BlockSpec((1, BN, HALF), lambda i: (1, i, 0)),
            pl.BlockSpec((BN, HALF), lambda i: (i, 0)),
            pl.BlockSpec((BN, D), lambda i: (i, 0)),
            pl.BlockSpec((D, D), lambda i: (0, 0)),
            pl.BlockSpec((1, D), lambda i: (0, 0)),
            pl.BlockSpec((1, D), lambda i: (0, 0)),
            pl.BlockSpec((1, D), lambda i: (0, 0)),
        ],
        out_specs=pl.BlockSpec((BN, D), lambda i: (i, 0)),
        out_shape=jax.ShapeDtypeStruct((N, D), jnp.float32),
    )(agg2, agg2, deg2, x, W, b.reshape(1, D), gamma.reshape(1, D),
      beta.reshape(1, D))
    return out
